# Initial kernel scaffold; baseline (speedup 1.0000x reference)
import jax
import jax.numpy as jnp
from jax.experimental import pallas as pl


def kernel(node_s, node_v, edge_s, edge_v, edge_index, batch, params):
    raise NotImplementedError("write your pallas kernel here")


# speedup vs baseline: 2.4545x; 2.4545x over previous
import functools

import jax
import jax.numpy as jnp
from jax import lax
from jax.experimental import pallas as pl
from jax.experimental.pallas import tpu as pltpu
from jax.experimental.pallas import tpu_sc as plsc

NN = 50000
NE = 800000
NG = 1000
EPS = 1e-8
LN_EPS = 1e-5
HP = 384

F32 = jnp.float32


def _mm(a, w):
    return jnp.dot(a, w, preferred_element_type=F32)


def _ln(s, g, b):
    mu = jnp.mean(s, axis=1, keepdims=True)
    var = jnp.mean(jnp.square(s - mu), axis=1, keepdims=True)
    return (s - mu) * lax.rsqrt(var + LN_EPS) * g[None, :] + b[None, :]


def _vnorm3(vx, vy, vz):
    return jnp.sqrt(jnp.maximum(vx * vx + vy * vy + vz * vz, EPS))


BN = 400
BE = 640


def _embed_nodes_k(s_ref, v_ref, g_ref, b_ref, wh_ref, ws_ref, wsb_ref,
                   wv_ref, out_ref):
    s = _ln(s_ref[...], g_ref[...], b_ref[...])
    v = v_ref[...]
    q = jnp.maximum(jnp.sum(v * v, axis=1, keepdims=True), EPS)
    v = v * lax.rsqrt(q)
    wh = wh_ref[...]
    vh = [v[:, d:d + 1] * wh[None, :] for d in range(3)]
    vn = _vnorm3(*vh)
    ws = ws_ref[...]
    sout = _mm(s, ws[0:66]) + _mm(vn, ws[66:130]) + wsb_ref[...][None, :]
    out_ref[:, 0:128] = sout
    wv = wv_ref[...]
    for d in range(3):
        out_ref[:, 128 + 64 * d:128 + 64 * (d + 1)] = _mm(vh[d], wv)


def _embed_nodes(node_s, node_v3, p):
    wh = p['gvp_v']['wh'].reshape(64)
    return pl.pallas_call(
        _embed_nodes_k,
        grid=(NN // BN,),
        in_specs=[
            pl.BlockSpec((BN, 66), lambda i: (i, 0)),
            pl.BlockSpec((BN, 3), lambda i: (i, 0)),
            pl.BlockSpec((66,), lambda i: (0,)),
            pl.BlockSpec((66,), lambda i: (0,)),
            pl.BlockSpec((64,), lambda i: (0,)),
            pl.BlockSpec((130, 128), lambda i: (0, 0)),
            pl.BlockSpec((128,), lambda i: (0,)),
            pl.BlockSpec((64, 64), lambda i: (0, 0)),
        ],
        out_specs=pl.BlockSpec((BN, HP), lambda i: (i, 0)),
        out_shape=jax.ShapeDtypeStruct((NN, HP), F32),
    )(node_s, node_v3, p['ln_v']['g'], p['ln_v']['b'], wh,
      p['gvp_v']['ws_w'], p['gvp_v']['ws_b'], p['gvp_v']['wv'])


def _embed_edges_k(s_ref, v_ref, g_ref, b_ref, wh_ref, ws_ref, wsb_ref,
                   wv_ref, out_ref):
    s = _ln(s_ref[...], g_ref[...], b_ref[...])
    v = v_ref[...]
    q = jnp.maximum(jnp.sum(v * v, axis=1, keepdims=True), EPS)
    v = v * lax.rsqrt(q)
    wh0 = wh_ref[0]
    vh = v * wh0
    vn = jnp.sqrt(jnp.maximum(jnp.sum(vh * vh, axis=1, keepdims=True), EPS))
    ws = ws_ref[...]
    sout = _mm(s, ws[0:16]) + vn * ws[16][None, :] + wsb_ref[...][None, :]
    out_ref[:, 0:32] = sout
    out_ref[:, 32:35] = vh * wv_ref[0]
    out_ref[:, 35:36] = jnp.zeros_like(vn)


def _embed_edges(edge_s, edge_v3, p):
    B = 1000
    wh = p['gvp_e']['wh'].reshape(1)
    wv = p['gvp_e']['wv'].reshape(1)
    return pl.pallas_call(
        _embed_edges_k,
        grid=(NE // B,),
        in_specs=[
            pl.BlockSpec((B, 16), lambda i: (i, 0)),
            pl.BlockSpec((B, 3), lambda i: (i, 0)),
            pl.BlockSpec((16,), lambda i: (0,)),
            pl.BlockSpec((16,), lambda i: (0,)),
            pl.BlockSpec((1,), lambda i: (0,)),
            pl.BlockSpec((17, 32), lambda i: (0, 0)),
            pl.BlockSpec((32,), lambda i: (0,)),
            pl.BlockSpec((1,), lambda i: (0,)),
        ],
        out_specs=pl.BlockSpec((B, 36), lambda i: (i, 0)),
        out_shape=jax.ShapeDtypeStruct((NE, 36), F32),
    )(edge_s, edge_v3, p['ln_e']['g'], p['ln_e']['b'], wh,
      p['gvp_e']['ws_w'], p['gvp_e']['ws_b'], wv)


def _messages_k(gsrc, gdst, eft, whs, whm, whd, wa, wb, wc, wd, b0, wv0,
                wh1, ws1s, ws1n, b1, wv1, wh2, ws2s, ws2n, b2, wv2, out):
    ss = gsrc[:, 0:128]
    sd = gdst[:, 0:128]
    es = eft[:, 0:32]
    whm_row = whm[...]
    vh = []
    for d in range(3):
        vsd = gsrc[:, 128 + 64 * d:128 + 64 * (d + 1)]
        vdd = gdst[:, 128 + 64 * d:128 + 64 * (d + 1)]
        evd = eft[:, 32 + d:33 + d]
        vh.append(_mm(vsd, whs[...]) + _mm(vdd, whd[...]) + evd * whm_row)
    vn = _vnorm3(*vh)
    s0 = (_mm(ss, wa[...]) + _mm(es, wb[...]) + _mm(sd, wc[...])
          + _mm(vn, wd[...]) + b0[...][None, :])
    s0 = jnp.maximum(s0, 0.0)
    u = [_mm(vh[d], wv0[...]) for d in range(3)]
    gate = jax.nn.sigmoid(_vnorm3(*u))
    u = [u[d] * gate for d in range(3)]
    vh1 = [_mm(u[d], wh1[...]) for d in range(3)]
    vn1 = _vnorm3(*vh1)
    s1 = _mm(s0, ws1s[...]) + _mm(vn1, ws1n[...]) + b1[...][None, :]
    s1 = jnp.maximum(s1, 0.0)
    u1 = [_mm(vh1[d], wv1[...]) for d in range(3)]
    gate1 = jax.nn.sigmoid(_vnorm3(*u1))
    u1 = [u1[d] * gate1 for d in range(3)]
    vh2 = [_mm(u1[d], wh2[...]) for d in range(3)]
    vn2 = _vnorm3(*vh2)
    s2 = _mm(s1, ws2s[...]) + _mm(vn2, ws2n[...]) + b2[...][None, :]
    out[:, 0:128] = s2
    for d in range(3):
        out[:, 128 + 64 * d:128 + 64 * (d + 1)] = _mm(vh2[d], wv2[...])


def _messages(gsrc, gdst, eft, cp):
    m0, m1, m2 = cp['m0'], cp['m1'], cp['m2']
    ws0 = m0['ws_w']
    args = [
        m0['wh'][0:64], m0['wh'][64:65], m0['wh'][65:129],
        ws0[0:128], ws0[128:160], ws0[160:288], ws0[288:417], m0['ws_b'],
        m0['wv'],
        m1['wh'], m1['ws_w'][0:128], m1['ws_w'][128:192], m1['ws_b'],
        m1['wv'],
        m2['wh'], m2['ws_w'][0:128], m2['ws_w'][128:192], m2['ws_b'],
        m2['wv'],
    ]
    wspecs = []
    for a in args:
        nd = a.ndim
        wspecs.append(pl.BlockSpec(a.shape, (lambda i: (0, 0)) if nd == 2
                                   else (lambda i: (0,))))
    return pl.pallas_call(
        _messages_k,
        grid=(NE // BE,),
        in_specs=[
            pl.BlockSpec((BE, HP), lambda i: (i, 0)),
            pl.BlockSpec((BE, HP), lambda i: (i, 0)),
            pl.BlockSpec((BE, 36), lambda i: (i, 0)),
        ] + wspecs,
        out_specs=pl.BlockSpec((BE, HP), lambda i: (i, 0)),
        out_shape=jax.ShapeDtypeStruct((NE, HP), F32),
    )(gsrc, gdst, eft, *args)


def _node_update_k(h, agg, cnt, g0, bb0, wh, wss, wsn, b, wv,
                   wh1, ws1s, ws1n, b1, wv1, g1, bb1, out):
    rec = 1.0 / jnp.maximum(cnt[:, 0:1], 1.0)
    s = h[:, 0:128] + agg[:, 0:128] * rec
    v = [h[:, 128 + 64 * d:128 + 64 * (d + 1)]
         + agg[:, 128 + 64 * d:128 + 64 * (d + 1)] * rec for d in range(3)]
    s = _ln(s, g0[...], bb0[...])
    nsq = jnp.maximum(v[0] ** 2 + v[1] ** 2 + v[2] ** 2, EPS)
    q = jnp.mean(nsq, axis=1, keepdims=True)
    ir = lax.rsqrt(q)
    v = [v[d] * ir for d in range(3)]
    vh = [_mm(v[d], wh[...]) for d in range(3)]
    vn = _vnorm3(*vh)
    sh = _mm(s, wss[...]) + _mm(vn, wsn[...]) + b[...][None, :]
    sh = jnp.maximum(sh, 0.0)
    u = [_mm(vh[d], wv[...]) for d in range(3)]
    gate = jax.nn.sigmoid(_vnorm3(*u))
    u = [u[d] * gate for d in range(3)]
    vh1 = [_mm(u[d], wh1[...]) for d in range(3)]
    vn1 = _vnorm3(*vh1)
    ds_ = _mm(sh, ws1s[...]) + _mm(vn1, ws1n[...]) + b1[...][None, :]
    dv = [_mm(vh1[d], wv1[...]) for d in range(3)]
    s2 = _ln(s + ds_, g1[...], bb1[...])
    v2 = [v[d] + dv[d] for d in range(3)]
    nsq2 = jnp.maximum(v2[0] ** 2 + v2[1] ** 2 + v2[2] ** 2, EPS)
    q2 = jnp.mean(nsq2, axis=1, keepdims=True)
    ir2 = lax.rsqrt(q2)
    out[:, 0:128] = s2
    for d in range(3):
        out[:, 128 + 64 * d:128 + 64 * (d + 1)] = v2[d] * ir2


def _node_update(h, agg, cnt, lp):
    ff0, ff1 = lp['ff0'], lp['ff1']
    args = [
        lp['norm0']['g'], lp['norm0']['b'],
        ff0['wh'], ff0['ws_w'][0:128], ff0['ws_w'][128:256], ff0['ws_b'],
        ff0['wv'],
        ff1['wh'], ff1['ws_w'][0:512], ff1['ws_w'][512:640], ff1['ws_b'],
        ff1['wv'],
        lp['norm1']['g'], lp['norm1']['b'],
    ]
    wspecs = []
    for a in args:
        nd = a.ndim
        wspecs.append(pl.BlockSpec(a.shape, (lambda i: (0, 0)) if nd == 2
                                   else (lambda i: (0,))))
    return pl.pallas_call(
        _node_update_k,
        grid=(NN // BN,),
        in_specs=[
            pl.BlockSpec((BN, HP), lambda i: (i, 0)),
            pl.BlockSpec((BN, HP), lambda i: (i, 0)),
            pl.BlockSpec((BN, 128), lambda i: (i, 0)),
        ] + wspecs,
        out_specs=pl.BlockSpec((BN, HP), lambda i: (i, 0)),
        out_shape=jax.ShapeDtypeStruct((NN, HP), F32),
    )(h, agg, cnt, *args)


def _readout_k(h, g, bb, wh, wss, wsn, b, out):
    s = _ln(h[:, 0:128], g[...], bb[...])
    v = [h[:, 128 + 64 * d:128 + 64 * (d + 1)] for d in range(3)]
    nsq = jnp.maximum(v[0] ** 2 + v[1] ** 2 + v[2] ** 2, EPS)
    q = jnp.mean(nsq, axis=1, keepdims=True)
    ir = lax.rsqrt(q)
    v = [v[d] * ir for d in range(3)]
    vh = [_mm(v[d], wh[...]) for d in range(3)]
    vn = _vnorm3(*vh)
    o = _mm(s, wss[...]) + _mm(vn, wsn[...]) + b[...][None, :]
    out[...] = jnp.maximum(o, 0.0)


def _readout(h, p):
    go = p['gvp_out']
    args = [p['ln_out']['g'], p['ln_out']['b'], go['wh'],
            go['ws_w'][0:128], go['ws_w'][128:192], go['ws_b']]
    wspecs = []
    for a in args:
        nd = a.ndim
        wspecs.append(pl.BlockSpec(a.shape, (lambda i: (0, 0)) if nd == 2
                                   else (lambda i: (0,))))
    return pl.pallas_call(
        _readout_k,
        grid=(NN // BN,),
        in_specs=[pl.BlockSpec((BN, HP), lambda i: (i, 0))] + wspecs,
        out_specs=pl.BlockSpec((BN, 128), lambda i: (i, 0)),
        out_shape=jax.ShapeDtypeStruct((NN, 128), F32),
    )(h, *args)


_MESH = dict(core_axis_name="c", subcore_axis_name="s", num_cores=2,
             num_subcores=16)

GCH = 40
GW = NE // 32 // GCH


def _gather_body(h_hbm, src_hbm, dst_hbm, osrc_hbm, odst_hbm,
                 idx8s, idx8d, bufs, gsem, wsem0, wsem1):
    c = lax.axis_index("c")
    s = lax.axis_index("s")
    wid = s * 2 + c
    ebase = wid * (NE // 32)
    wsems = (wsem0, wsem1)

    def chunk(j, p, wsem, idxrow, first):
        a, b = bufs[2 * p], bufs[2 * p + 1]
        rows = pl.ds(ebase + j * GCH, GCH)

        def drain():
            pltpu.make_async_copy(a, osrc_hbm.at[rows], wsem).wait()
            pltpu.make_async_copy(b, odst_hbm.at[rows], wsem).wait()
        if first is None:
            drain()
        else:
            pl.when(jnp.logical_not(first))(drain)

        ga = pltpu.async_copy(h_hbm.at[idx8s.at[idxrow]], a, gsem)
        gb = pltpu.async_copy(h_hbm.at[idx8d.at[idxrow]], b, gsem)
        ga.wait()
        gb.wait()
        pltpu.async_copy(a, osrc_hbm.at[rows], wsem)
        pltpu.async_copy(b, odst_hbm.at[rows], wsem)

    def group(g, carry):
        pltpu.sync_copy(src_hbm.at[wid, pl.ds(g * 8, 8)], idx8s)
        pltpu.sync_copy(dst_hbm.at[wid, pl.ds(g * 8, 8)], idx8d)
        for r in range(8):
            j = g * 8 + r
            first = (g == 0) if r < 2 else None
            chunk(j, r % 2, wsems[r % 2], r, first)
        return carry

    lax.fori_loop(0, GW // 8, group, 0)
    pltpu.sync_copy(src_hbm.at[wid, pl.ds(GW - 1, 1)],
                    idx8s.at[pl.ds(0, 1)])
    pltpu.sync_copy(dst_hbm.at[wid, pl.ds(GW - 1, 1)],
                    idx8d.at[pl.ds(0, 1)])
    chunk(GW - 1, 0, wsem0, 0, None)
    for wsem in wsems:
        for _ in range(2):
            pltpu.make_async_copy(bufs[0], osrc_hbm.at[pl.ds(0, GCH)],
                                  wsem).wait()


def _gather(h, srcI, dstI):
    f = pl.kernel(
        _gather_body,
        out_type=(jax.ShapeDtypeStruct((NE, HP), F32),
                  jax.ShapeDtypeStruct((NE, HP), F32)),
        mesh=plsc.VectorSubcoreMesh(**_MESH),
        scratch_types=[
            pltpu.VMEM((8, GCH), jnp.int32),
            pltpu.VMEM((8, GCH), jnp.int32),
            [pltpu.VMEM((GCH, HP), F32) for _ in range(4)],
            pltpu.SemaphoreType.DMA,
            pltpu.SemaphoreType.DMA,
            pltpu.SemaphoreType.DMA,
        ],
    )
    return f(h, srcI, dstI)


SCH = 80
NCHK = (NE // 16) // SCH
NQ = 12544
AROW = 12672
_PASSES = (
    [(0, 0), (0, 1), (0, 2), (0, 3), (256, 0), (256, 1)],
    [(128, 0), (128, 1), (128, 2), (128, 3), (256, 2), (256, 3)],
)
_VOFF = (0, 16, 32, 48, 64)


def _zfill(zbuf, nrow):
    def zrow(i, carry):
        for k in range(8):
            zbuf[i, pl.ds(k * 16, 16)] = jnp.zeros((16,), F32)
        return carry
    lax.fori_loop(0, nrow, zrow, 0)


def _zero_acc(zbuf, acc, tid):
    for z in range(33):
        pltpu.sync_copy(zbuf, acc.at[pl.ds(tid * 792 + z * 24, 24)])


def _cidx(dstG, idxb, r, base):
    for k in _VOFF:
        d = dstG[r, pl.ds(k, 16)]
        inb = (d >= base) & (d < base + NQ)
        idxb[0, pl.ds(k, 16)] = jnp.where(inb, d - base, NQ + (d & 127))


def _scatter_body(msg_hbm, dst_hbm, agg_hbm, acc, dstG, idxb, mb0, mb1,
                  lsem0, lsem1, ssem):
    c = lax.axis_index("c")
    tid = lax.axis_index("s")
    erow = tid * (NE // 16)
    mbs = (mb0, mb1)
    lsems = (lsem0, lsem1)

    def load(j, col, p):
        pltpu.async_copy(
            msg_hbm.at[pl.ds(erow + j * SCH, SCH), pl.ds(col, 128)],
            mbs[p], lsems[p])

    def waitload(col, p):
        pltpu.make_async_copy(
            msg_hbm.at[pl.ds(erow, SCH), pl.ds(col, 128)],
            mbs[p], lsems[p]).wait()

    def do_pass(col, q):
        base = q * NQ
        _zfill(mb1, 24)
        _zero_acc(mb1.at[pl.ds(0, 24)], acc, tid)
        plsc.subcore_barrier()
        load(0, col, 0)

        def chunk(j, r, p):
            @pl.when(j + 1 < NCHK)
            def _():
                load(j + 1, col, 1 - p)
            waitload(col, p)
            _cidx(dstG, idxb, r, base)
            pltpu.async_copy(mbs[p].at[pl.ds(0, SCH)],
                             acc.at[idxb.at[0]], ssem, add=True).wait()

        def grp(g, carry):
            pltpu.sync_copy(dst_hbm.at[tid, pl.ds(g * 8, 8)], dstG)
            for r in range(8):
                j = g * 8 + r
                chunk(j, r, r % 2)
            return carry

        lax.fori_loop(0, NCHK // 8, grp, 0)
        pltpu.sync_copy(dst_hbm.at[tid, pl.ds(NCHK - 1, 1)],
                        dstG.at[pl.ds(0, 1)])
        chunk(NCHK - 1, 0, (NCHK - 1) % 2)
        plsc.subcore_barrier()
        _flush_quarter(acc, agg_hbm, tid, q, col, 128)

    for p in range(6):
        c0, q0 = _PASSES[0][p]
        c1, q1 = _PASSES[1][p]
        pl.when(c == 0)(lambda a=c0, d=q0: do_pass(a, d))
        pl.when(c == 1)(lambda a=c1, d=q1: do_pass(a, d))


def _scatter(msgs, dst3):
    f = pl.kernel(
        _scatter_body,
        out_type=jax.ShapeDtypeStruct((NN, HP), F32),
        mesh=plsc.VectorSubcoreMesh(**_MESH),
        scratch_types=[
            pltpu.VMEM_SHARED((AROW, 128), F32),
            pltpu.VMEM((8, SCH), jnp.int32),
            pltpu.VMEM((1, SCH), jnp.int32),
            pltpu.VMEM((SCH, 128), F32),
            pltpu.VMEM((SCH, 128), F32),
            pltpu.SemaphoreType.DMA,
            pltpu.SemaphoreType.DMA,
            pltpu.SemaphoreType.DMA,
        ],
    )
    return f(msgs, dst3)


def _flush_quarter(acc, out_hbm, tid, q, col, ncol):
    base = q * NQ
    if q == 3:
        pl.when(tid < 15)(lambda: pltpu.sync_copy(
            acc.at[pl.ds(tid * 784, 784), pl.ds(0, ncol)],
            out_hbm.at[pl.ds(base + tid * 784, 784), pl.ds(col, ncol)]))
        pl.when(tid == 15)(lambda: pltpu.sync_copy(
            acc.at[pl.ds(15 * 784, 608), pl.ds(0, ncol)],
            out_hbm.at[pl.ds(base + 15 * 784, 608), pl.ds(col, ncol)]))
    else:
        pltpu.sync_copy(
            acc.at[pl.ds(tid * 784, 784), pl.ds(0, ncol)],
            out_hbm.at[pl.ds(base + tid * 784, 784), pl.ds(col, ncol)])


def _cnt_body(dst_hbm, cnt_hbm, acc, dstG, idxb, ones, zbuf, ssem):
    c = lax.axis_index("c")
    tid = lax.axis_index("s")
    _zfill(zbuf, 24)

    def ofill(i, carry):
        for k in range(8):
            ones[i, pl.ds(k * 16, 16)] = jnp.ones((16,), F32)
        return carry
    lax.fori_loop(0, SCH, ofill, 0)

    def do_pass(q):
        base = q * NQ
        _zero_acc(zbuf, acc, tid)
        plsc.subcore_barrier()

        def chunk(j, r):
            _cidx(dstG, idxb, r, base)
            pltpu.async_copy(ones, acc.at[idxb.at[0]], ssem, add=True).wait()

        def grp(g, carry):
            pltpu.sync_copy(dst_hbm.at[tid, pl.ds(g * 8, 8)], dstG)
            for r in range(8):
                chunk(g * 8 + r, r)
            return carry

        lax.fori_loop(0, NCHK // 8, grp, 0)
        pltpu.sync_copy(dst_hbm.at[tid, pl.ds(NCHK - 1, 1)],
                        dstG.at[pl.ds(0, 1)])
        chunk(NCHK - 1, 0)
        plsc.subcore_barrier()
        _flush_quarter(acc, cnt_hbm, tid, q, 0, 128)

    for p in range(2):
        pl.when(c == 0)(lambda q=p: do_pass(q))
        pl.when(c == 1)(lambda q=2 + p: do_pass(q))


def _cnt(dst3):
    f = pl.kernel(
        _cnt_body,
        out_type=jax.ShapeDtypeStruct((NN, 128), F32),
        mesh=plsc.VectorSubcoreMesh(**_MESH),
        scratch_types=[
            pltpu.VMEM_SHARED((AROW, 128), F32),
            pltpu.VMEM((8, SCH), jnp.int32),
            pltpu.VMEM((1, SCH), jnp.int32),
            pltpu.VMEM((SCH, 128), F32),
            pltpu.VMEM((24, 128), F32),
            pltpu.SemaphoreType.DMA,
        ],
    )
    return f(dst3)


def _pool_body(x_hbm, bat_hbm, out_hbm, acc, batb, rb, zbuf, ssem):
    c = lax.axis_index("c")
    tid = lax.axis_index("s")

    @pl.when((c == 0) & (tid == 0))
    def _():
        _zfill(zbuf, 125)
        for z in range(8):
            pltpu.sync_copy(zbuf, acc.at[pl.ds(z * 125, 125)])

    plsc.subcore_barrier()

    @pl.when(c == 0)
    def _():
        def chunk(cid):
            pltpu.sync_copy(bat_hbm.at[cid], batb)
            pltpu.sync_copy(x_hbm.at[pl.ds(cid * 200, 200)], rb)
            fires = []
            for i in range(2):
                fires.append(pltpu.async_copy(
                    rb.at[pl.ds(i * 100, 100)],
                    acc.at[batb.at[i]], ssem, add=True))
            for fcp in fires:
                fcp.wait()

        def body(k, carry):
            chunk(k * 16 + tid)
            return carry

        lax.fori_loop(0, 15, body, 0)

        @pl.when(tid < 10)
        def _():
            chunk(240 + tid)

    plsc.subcore_barrier()

    @pl.when((c == 0) & (tid == 0))
    def _():
        pltpu.sync_copy(acc, out_hbm)


def _pool(x, bat3):
    f = pl.kernel(
        _pool_body,
        out_type=jax.ShapeDtypeStruct((NG, 128), F32),
        mesh=plsc.VectorSubcoreMesh(**_MESH),
        scratch_types=[
            pltpu.VMEM_SHARED((NG, 128), F32),
            pltpu.VMEM((2, 100), jnp.int32),
            pltpu.VMEM((200, 128), F32),
            pltpu.VMEM((125, 128), F32),
            pltpu.SemaphoreType.DMA,
        ],
    )
    return f(x, bat3)


def kernel(node_s, node_v, edge_s, edge_v, edge_index, batch, params):
    src = edge_index[0]
    dst = edge_index[1]
    srcI = src.reshape(32, GW, GCH)
    dstI = dst.reshape(32, GW, GCH)
    dst3 = dst.reshape(16, NCHK, SCH)
    bat3 = batch.reshape(250, 2, 100)
    nv3 = node_v.reshape(NN, 3)
    ev3 = edge_v.reshape(NE, 3)

    h = _embed_nodes(node_s, nv3, params)
    eft = _embed_edges(edge_s, ev3, params)
    cnt = _cnt(dst3)
    for i in range(3):
        lp = params['layers'][i]
        gs, gd = _gather(h, srcI, dstI)
        msgs = _messages(gs, gd, eft, lp['conv'])
        agg = _scatter(msgs, dst3)
        h = _node_update(h, agg, cnt, lp)
    out50 = _readout(h, params)
    return _pool(out50, bat3)


# speedup vs baseline: 2.5250x; 1.0287x over previous
import jax
import jax.numpy as jnp
from jax import lax
from jax.experimental import pallas as pl
from jax.experimental.pallas import tpu as pltpu
from jax.experimental.pallas import tpu_sc as plsc

NN = 50000
NE = 800000
NG = 1000
EPS = 1e-8
LN_EPS = 1e-5
HP = 384

F32 = jnp.float32


def _mm(a, w):
    return jnp.dot(a, w, preferred_element_type=F32)


def _mmb(a, w):
    return jnp.dot(a.astype(jnp.bfloat16), w.astype(jnp.bfloat16),
                   preferred_element_type=F32)


def _ln(s, g, b):
    mu = jnp.mean(s, axis=1, keepdims=True)
    var = jnp.mean(jnp.square(s - mu), axis=1, keepdims=True)
    return (s - mu) * lax.rsqrt(var + LN_EPS) * g[None, :] + b[None, :]


def _vnorm3(vx, vy, vz):
    return jnp.sqrt(jnp.maximum(vx * vx + vy * vy + vz * vz, EPS))


BN = 400
BE = 640


def _embed_nodes_k(s_ref, v_ref, g_ref, b_ref, wh_ref, ws_ref, wsb_ref,
                   wv_ref, out_ref):
    s = _ln(s_ref[...], g_ref[...], b_ref[...])
    v = v_ref[:, 0, :]
    q = jnp.maximum(jnp.sum(v * v, axis=1, keepdims=True), EPS)
    v = v * lax.rsqrt(q)
    wh = wh_ref[...]
    vh = [v[:, d:d + 1] * wh[None, :] for d in range(3)]
    vn = _vnorm3(*vh)
    ws = ws_ref[...]
    sout = _mm(s, ws[0:66]) + _mm(vn, ws[66:130]) + wsb_ref[...][None, :]
    out_ref[:, 0:128] = sout
    wv = wv_ref[...]
    for d in range(3):
        out_ref[:, 128 + 64 * d:128 + 64 * (d + 1)] = _mm(vh[d], wv)


def _embed_nodes(node_s, node_v, p):
    wh = p['gvp_v']['wh'].reshape(64)
    return pl.pallas_call(
        _embed_nodes_k,
        grid=(NN // BN,),
        in_specs=[
            pl.BlockSpec((BN, 66), lambda i: (i, 0)),
            pl.BlockSpec((BN, 1, 3), lambda i: (i, 0, 0)),
            pl.BlockSpec((66,), lambda i: (0,)),
            pl.BlockSpec((66,), lambda i: (0,)),
            pl.BlockSpec((64,), lambda i: (0,)),
            pl.BlockSpec((130, 128), lambda i: (0, 0)),
            pl.BlockSpec((128,), lambda i: (0,)),
            pl.BlockSpec((64, 64), lambda i: (0, 0)),
        ],
        out_specs=pl.BlockSpec((BN, HP), lambda i: (i, 0)),
        out_shape=jax.ShapeDtypeStruct((NN, HP), F32),
    )(node_s, node_v, p['ln_v']['g'], p['ln_v']['b'], wh,
      p['gvp_v']['ws_w'], p['gvp_v']['ws_b'], p['gvp_v']['wv'])


def _embed_edges_k(s_ref, v_ref, g_ref, b_ref, wh_ref, ws_ref, wsb_ref,
                   wv_ref, out_ref):
    s = _ln(s_ref[...], g_ref[...], b_ref[...])
    v = v_ref[:, 0, :]
    q = jnp.maximum(jnp.sum(v * v, axis=1, keepdims=True), EPS)
    v = v * lax.rsqrt(q)
    wh0 = wh_ref[0]
    vh = v * wh0
    vn = jnp.sqrt(jnp.maximum(jnp.sum(vh * vh, axis=1, keepdims=True), EPS))
    ws = ws_ref[...]
    sout = _mm(s, ws[0:16]) + vn * ws[16][None, :] + wsb_ref[...][None, :]
    out_ref[:, 0:32] = sout
    out_ref[:, 32:35] = vh * wv_ref[0]
    out_ref[:, 35:36] = jnp.zeros_like(vn)


def _embed_edges(edge_s, edge_v, p):
    B = 1000
    wh = p['gvp_e']['wh'].reshape(1)
    wv = p['gvp_e']['wv'].reshape(1)
    return pl.pallas_call(
        _embed_edges_k,
        grid=(NE // B,),
        in_specs=[
            pl.BlockSpec((B, 16), lambda i: (i, 0)),
            pl.BlockSpec((B, 1, 3), lambda i: (i, 0, 0)),
            pl.BlockSpec((16,), lambda i: (0,)),
            pl.BlockSpec((16,), lambda i: (0,)),
            pl.BlockSpec((1,), lambda i: (0,)),
            pl.BlockSpec((17, 32), lambda i: (0, 0)),
            pl.BlockSpec((32,), lambda i: (0,)),
            pl.BlockSpec((1,), lambda i: (0,)),
        ],
        out_specs=pl.BlockSpec((B, 36), lambda i: (i, 0)),
        out_shape=jax.ShapeDtypeStruct((NE, 36), F32),
    )(edge_s, edge_v, p['ln_e']['g'], p['ln_e']['b'], wh,
      p['gvp_e']['ws_w'], p['gvp_e']['ws_b'], wv)


def _messages_k(gsrc, gdst, eft, whs, whm, whd, wa, wb, wc, wd, b0, wv0,
                wh1, ws1s, ws1n, b1, wv1, wh2, ws2s, ws2n, b2, wv2, out):
    ss = gsrc[:, 0:128]
    sd = gdst[:, 0:128]
    es = eft[:, 0:32]
    whm_row = whm[...]
    vh = []
    for d in range(3):
        vsd = gsrc[:, 128 + 64 * d:128 + 64 * (d + 1)]
        vdd = gdst[:, 128 + 64 * d:128 + 64 * (d + 1)]
        evd = eft[:, 32 + d:33 + d]
        vh.append(_mm(vsd, whs[...]) + _mm(vdd, whd[...]) + evd * whm_row)
    vn = _vnorm3(*vh)
    s0 = (_mm(ss, wa[...]) + _mm(es, wb[...]) + _mm(sd, wc[...])
          + _mm(vn, wd[...]) + b0[...][None, :])
    s0 = jnp.maximum(s0, 0.0)
    u = [_mm(vh[d], wv0[...]) for d in range(3)]
    gate = jax.nn.sigmoid(_vnorm3(*u))
    u = [u[d] * gate for d in range(3)]
    vh1 = [_mm(u[d], wh1[...]) for d in range(3)]
    vn1 = _vnorm3(*vh1)
    s1 = _mm(s0, ws1s[...]) + _mm(vn1, ws1n[...]) + b1[...][None, :]
    s1 = jnp.maximum(s1, 0.0)
    u1 = [_mm(vh1[d], wv1[...]) for d in range(3)]
    gate1 = jax.nn.sigmoid(_vnorm3(*u1))
    u1 = [u1[d] * gate1 for d in range(3)]
    vh2 = [_mm(u1[d], wh2[...]) for d in range(3)]
    vn2 = _vnorm3(*vh2)
    s2 = _mm(s1, ws2s[...]) + _mm(vn2, ws2n[...]) + b2[...][None, :]
    out[:, 0:128] = s2
    for d in range(3):
        out[:, 128 + 64 * d:128 + 64 * (d + 1)] = _mm(vh2[d], wv2[...])


def _messages(gsrc, gdst, eft, cp):
    m0, m1, m2 = cp['m0'], cp['m1'], cp['m2']
    ws0 = m0['ws_w']
    args = [
        m0['wh'][0:64], m0['wh'][64:65], m0['wh'][65:129],
        ws0[0:128], ws0[128:160], ws0[160:288], ws0[288:417], m0['ws_b'],
        m0['wv'],
        m1['wh'], m1['ws_w'][0:128], m1['ws_w'][128:192], m1['ws_b'],
        m1['wv'],
        m2['wh'], m2['ws_w'][0:128], m2['ws_w'][128:192], m2['ws_b'],
        m2['wv'],
    ]
    wspecs = []
    for a in args:
        nd = a.ndim
        wspecs.append(pl.BlockSpec(a.shape, (lambda i: (0, 0)) if nd == 2
                                   else (lambda i: (0,))))
    return pl.pallas_call(
        _messages_k,
        grid=(NE // BE,),
        in_specs=[
            pl.BlockSpec((BE, HP), lambda i: (i, 0)),
            pl.BlockSpec((BE, HP), lambda i: (i, 0)),
            pl.BlockSpec((BE, 36), lambda i: (i, 0)),
        ] + wspecs,
        out_specs=pl.BlockSpec((BE, HP), lambda i: (i, 0)),
        out_shape=jax.ShapeDtypeStruct((NE, HP), F32),
    )(gsrc, gdst, eft, *args)


def _node_update_k(h, agg, cnt, g0, bb0, wh, wss, wsn, b, wv,
                   wh1, ws1s, ws1n, b1, wv1, g1, bb1, out):
    rec = 1.0 / jnp.maximum(cnt[:, 0:1], 1.0)
    s = h[:, 0:128] + agg[:, 0:128] * rec
    v = [h[:, 128 + 64 * d:128 + 64 * (d + 1)]
         + agg[:, 128 + 64 * d:128 + 64 * (d + 1)] * rec for d in range(3)]
    s = _ln(s, g0[...], bb0[...])
    nsq = jnp.maximum(v[0] ** 2 + v[1] ** 2 + v[2] ** 2, EPS)
    q = jnp.mean(nsq, axis=1, keepdims=True)
    ir = lax.rsqrt(q)
    v = [v[d] * ir for d in range(3)]
    vh = [_mm(v[d], wh[...]) for d in range(3)]
    vn = _vnorm3(*vh)
    sh = _mm(s, wss[...]) + _mm(vn, wsn[...]) + b[...][None, :]
    sh = jnp.maximum(sh, 0.0)
    u = [_mm(vh[d], wv[...]) for d in range(3)]
    gate = jax.nn.sigmoid(_vnorm3(*u))
    u = [u[d] * gate for d in range(3)]
    vh1 = [_mm(u[d], wh1[...]) for d in range(3)]
    vn1 = _vnorm3(*vh1)
    ds_ = _mm(sh, ws1s[...]) + _mm(vn1, ws1n[...]) + b1[...][None, :]
    dv = [_mm(vh1[d], wv1[...]) for d in range(3)]
    s2 = _ln(s + ds_, g1[...], bb1[...])
    v2 = [v[d] + dv[d] for d in range(3)]
    nsq2 = jnp.maximum(v2[0] ** 2 + v2[1] ** 2 + v2[2] ** 2, EPS)
    q2 = jnp.mean(nsq2, axis=1, keepdims=True)
    ir2 = lax.rsqrt(q2)
    out[:, 0:128] = s2
    for d in range(3):
        out[:, 128 + 64 * d:128 + 64 * (d + 1)] = v2[d] * ir2


def _node_update(h, agg, cnt, lp):
    ff0, ff1 = lp['ff0'], lp['ff1']
    args = [
        lp['norm0']['g'], lp['norm0']['b'],
        ff0['wh'], ff0['ws_w'][0:128], ff0['ws_w'][128:256], ff0['ws_b'],
        ff0['wv'],
        ff1['wh'], ff1['ws_w'][0:512], ff1['ws_w'][512:640], ff1['ws_b'],
        ff1['wv'],
        lp['norm1']['g'], lp['norm1']['b'],
    ]
    wspecs = []
    for a in args:
        nd = a.ndim
        wspecs.append(pl.BlockSpec(a.shape, (lambda i: (0, 0)) if nd == 2
                                   else (lambda i: (0,))))
    return pl.pallas_call(
        _node_update_k,
        grid=(NN // BN,),
        in_specs=[
            pl.BlockSpec((BN, HP), lambda i: (i, 0)),
            pl.BlockSpec((BN, HP), lambda i: (i, 0)),
            pl.BlockSpec((BN, 128), lambda i: (i, 0)),
        ] + wspecs,
        out_specs=pl.BlockSpec((BN, HP), lambda i: (i, 0)),
        out_shape=jax.ShapeDtypeStruct((NN, HP), F32),
    )(h, agg, cnt, *args)


def _readout_k(h, g, bb, wh, wss, wsn, b, out):
    s = _ln(h[:, 0:128], g[...], bb[...])
    v = [h[:, 128 + 64 * d:128 + 64 * (d + 1)] for d in range(3)]
    nsq = jnp.maximum(v[0] ** 2 + v[1] ** 2 + v[2] ** 2, EPS)
    q = jnp.mean(nsq, axis=1, keepdims=True)
    ir = lax.rsqrt(q)
    v = [v[d] * ir for d in range(3)]
    vh = [_mm(v[d], wh[...]) for d in range(3)]
    vn = _vnorm3(*vh)
    o = _mm(s, wss[...]) + _mm(vn, wsn[...]) + b[...][None, :]
    out[...] = jnp.maximum(o, 0.0)


def _readout(h, p):
    go = p['gvp_out']
    args = [p['ln_out']['g'], p['ln_out']['b'], go['wh'],
            go['ws_w'][0:128], go['ws_w'][128:192], go['ws_b']]
    wspecs = []
    for a in args:
        nd = a.ndim
        wspecs.append(pl.BlockSpec(a.shape, (lambda i: (0, 0)) if nd == 2
                                   else (lambda i: (0,))))
    return pl.pallas_call(
        _readout_k,
        grid=(NN // BN,),
        in_specs=[pl.BlockSpec((BN, HP), lambda i: (i, 0))] + wspecs,
        out_specs=pl.BlockSpec((BN, 128), lambda i: (i, 0)),
        out_shape=jax.ShapeDtypeStruct((NN, 128), F32),
    )(h, *args)


_MESH = dict(core_axis_name="c", subcore_axis_name="s", num_cores=2,
             num_subcores=16)

GCH = 40
GW = NE // 32 // GCH


def _gather_body(h_hbm, src_hbm, dst_hbm, osrc_hbm, odst_hbm,
                 idx8s, idx8d, bufs, gsem, wsem0, wsem1):
    c = lax.axis_index("c")
    s = lax.axis_index("s")
    wid = s * 2 + c
    ebase = wid * (NE // 32)
    wsems = (wsem0, wsem1)

    def chunk(j, p, wsem, idxrow, first):
        a, b = bufs[2 * p], bufs[2 * p + 1]
        rows = pl.ds(ebase + j * GCH, GCH)
        isl = pl.ds(idxrow * GCH, GCH)

        def drain():
            pltpu.make_async_copy(a, osrc_hbm.at[rows], wsem).wait()
            pltpu.make_async_copy(b, odst_hbm.at[rows], wsem).wait()
        if first is None:
            drain()
        else:
            pl.when(jnp.logical_not(first))(drain)

        ga = pltpu.async_copy(h_hbm.at[idx8s.at[isl]], a, gsem)
        gb = pltpu.async_copy(h_hbm.at[idx8d.at[isl]], b, gsem)
        ga.wait()
        gb.wait()
        pltpu.async_copy(a, osrc_hbm.at[rows], wsem)
        pltpu.async_copy(b, odst_hbm.at[rows], wsem)

    def group(g, carry):
        pltpu.sync_copy(src_hbm.at[pl.ds(ebase + g * 8 * GCH, 8 * GCH)],
                        idx8s)
        pltpu.sync_copy(dst_hbm.at[pl.ds(ebase + g * 8 * GCH, 8 * GCH)],
                        idx8d)
        for r in range(8):
            j = g * 8 + r
            first = (g == 0) if r < 2 else None
            chunk(j, r % 2, wsems[r % 2], r, first)
        return carry

    lax.fori_loop(0, GW // 8, group, 0)
    pltpu.sync_copy(src_hbm.at[pl.ds(ebase + (GW - 1) * GCH, GCH)],
                    idx8s.at[pl.ds(0, GCH)])
    pltpu.sync_copy(dst_hbm.at[pl.ds(ebase + (GW - 1) * GCH, GCH)],
                    idx8d.at[pl.ds(0, GCH)])
    chunk(GW - 1, 0, wsem0, 0, None)
    for wsem in wsems:
        for _ in range(2):
            pltpu.make_async_copy(bufs[0], osrc_hbm.at[pl.ds(0, GCH)],
                                  wsem).wait()


def _gather(h, srcI, dstI):
    f = pl.kernel(
        _gather_body,
        out_type=(jax.ShapeDtypeStruct((NE, HP), F32),
                  jax.ShapeDtypeStruct((NE, HP), F32)),
        mesh=plsc.VectorSubcoreMesh(**_MESH),
        scratch_types=[
            pltpu.VMEM((8 * GCH,), jnp.int32),
            pltpu.VMEM((8 * GCH,), jnp.int32),
            [pltpu.VMEM((GCH, HP), F32) for _ in range(4)],
            pltpu.SemaphoreType.DMA,
            pltpu.SemaphoreType.DMA,
            pltpu.SemaphoreType.DMA,
        ],
    )
    return f(h, srcI, dstI)


SCH = 80
NCHK = (NE // 16) // SCH
NQ = 12544
AROW = 12672
_PASSES = (
    [(0, 0), (0, 1), (0, 2), (0, 3), (256, 0), (256, 1)],
    [(128, 0), (128, 1), (128, 2), (128, 3), (256, 2), (256, 3)],
)
_VOFF = (0, 16, 32, 48, 64)


def _zfill(zbuf, nrow):
    def zrow(i, carry):
        for k in range(8):
            zbuf[i, pl.ds(k * 16, 16)] = jnp.zeros((16,), F32)
        return carry
    lax.fori_loop(0, nrow, zrow, 0)


def _zero_acc(zbuf, acc, tid):
    for z in range(33):
        pltpu.sync_copy(zbuf, acc.at[pl.ds(tid * 792 + z * 24, 24)])


def _cidx(dstG, idxb, r, base):
    for k in _VOFF:
        d = dstG[pl.ds(r * SCH + k, 16)]
        inb = (d >= base) & (d < base + NQ)
        idxb[0, pl.ds(k, 16)] = jnp.where(inb, d - base, NQ + (d & 127))


def _scatter_body(msg_hbm, dst_hbm, agg_hbm, acc, dstG, idxb, mb0, mb1,
                  lsem0, lsem1, ssem):
    c = lax.axis_index("c")
    tid = lax.axis_index("s")
    erow = tid * (NE // 16)
    mbs = (mb0, mb1)
    lsems = (lsem0, lsem1)

    def load(j, col, p):
        pltpu.async_copy(
            msg_hbm.at[pl.ds(erow + j * SCH, SCH), pl.ds(col, 128)],
            mbs[p], lsems[p])

    def waitload(col, p):
        pltpu.make_async_copy(
            msg_hbm.at[pl.ds(erow, SCH), pl.ds(col, 128)],
            mbs[p], lsems[p]).wait()

    def do_pass(col, q):
        base = q * NQ
        _zfill(mb1, 24)
        _zero_acc(mb1.at[pl.ds(0, 24)], acc, tid)
        plsc.subcore_barrier()
        load(0, col, 0)

        def chunk(j, r, p):
            @pl.when(j + 1 < NCHK)
            def _():
                load(j + 1, col, 1 - p)
            waitload(col, p)
            _cidx(dstG, idxb, r, base)
            pltpu.async_copy(mbs[p].at[pl.ds(0, SCH)],
                             acc.at[idxb.at[0]], ssem, add=True).wait()

        def grp(g, carry):
            pltpu.sync_copy(
                dst_hbm.at[pl.ds(erow + g * 8 * SCH, 8 * SCH)], dstG)
            for r in range(8):
                j = g * 8 + r
                chunk(j, r, r % 2)
            return carry

        lax.fori_loop(0, NCHK // 8, grp, 0)
        pltpu.sync_copy(dst_hbm.at[pl.ds(erow + (NCHK - 1) * SCH, SCH)],
                        dstG.at[pl.ds(0, SCH)])
        chunk(NCHK - 1, 0, (NCHK - 1) % 2)
        plsc.subcore_barrier()
        _flush_quarter(acc, agg_hbm, tid, q, col, 128)

    for p in range(6):
        c0, q0 = _PASSES[0][p]
        c1, q1 = _PASSES[1][p]
        pl.when(c == 0)(lambda a=c0, d=q0: do_pass(a, d))
        pl.when(c == 1)(lambda a=c1, d=q1: do_pass(a, d))


def _scatter(msgs, dst1):
    f = pl.kernel(
        _scatter_body,
        out_type=jax.ShapeDtypeStruct((NN, HP), F32),
        mesh=plsc.VectorSubcoreMesh(**_MESH),
        scratch_types=[
            pltpu.VMEM_SHARED((AROW, 128), F32),
            pltpu.VMEM((8 * SCH,), jnp.int32),
            pltpu.VMEM((1, SCH), jnp.int32),
            pltpu.VMEM((SCH, 128), F32),
            pltpu.VMEM((SCH, 128), F32),
            pltpu.SemaphoreType.DMA,
            pltpu.SemaphoreType.DMA,
            pltpu.SemaphoreType.DMA,
        ],
    )
    return f(msgs, dst1)


def _flush_quarter(acc, out_hbm, tid, q, col, ncol):
    base = q * NQ
    if q == 3:
        pl.when(tid < 15)(lambda: pltpu.sync_copy(
            acc.at[pl.ds(tid * 784, 784), pl.ds(0, ncol)],
            out_hbm.at[pl.ds(base + tid * 784, 784), pl.ds(col, ncol)]))
        pl.when(tid == 15)(lambda: pltpu.sync_copy(
            acc.at[pl.ds(15 * 784, 608), pl.ds(0, ncol)],
            out_hbm.at[pl.ds(base + 15 * 784, 608), pl.ds(col, ncol)]))
    else:
        pltpu.sync_copy(
            acc.at[pl.ds(tid * 784, 784), pl.ds(0, ncol)],
            out_hbm.at[pl.ds(base + tid * 784, 784), pl.ds(col, ncol)])


def _cnt_body(dst_hbm, cnt_hbm, acc, dstG, idxb, ones, zbuf, ssem):
    c = lax.axis_index("c")
    tid = lax.axis_index("s")
    erow = tid * (NE // 16)
    _zfill(zbuf, 24)

    def ofill(i, carry):
        for k in range(8):
            ones[i, pl.ds(k * 16, 16)] = jnp.ones((16,), F32)
        return carry
    lax.fori_loop(0, SCH, ofill, 0)

    def do_pass(q):
        base = q * NQ
        _zero_acc(zbuf, acc, tid)
        plsc.subcore_barrier()

        def chunk(j, r):
            _cidx(dstG, idxb, r, base)
            pltpu.async_copy(ones, acc.at[idxb.at[0]], ssem, add=True).wait()

        def grp(g, carry):
            pltpu.sync_copy(
                dst_hbm.at[pl.ds(erow + g * 8 * SCH, 8 * SCH)], dstG)
            for r in range(8):
                chunk(g * 8 + r, r)
            return carry

        lax.fori_loop(0, NCHK // 8, grp, 0)
        pltpu.sync_copy(dst_hbm.at[pl.ds(erow + (NCHK - 1) * SCH, SCH)],
                        dstG.at[pl.ds(0, SCH)])
        chunk(NCHK - 1, 0)
        plsc.subcore_barrier()
        _flush_quarter(acc, cnt_hbm, tid, q, 0, 128)

    for p in range(2):
        pl.when(c == 0)(lambda q=p: do_pass(q))
        pl.when(c == 1)(lambda q=2 + p: do_pass(q))


def _cnt(dst1):
    f = pl.kernel(
        _cnt_body,
        out_type=jax.ShapeDtypeStruct((NN, 128), F32),
        mesh=plsc.VectorSubcoreMesh(**_MESH),
        scratch_types=[
            pltpu.VMEM_SHARED((AROW, 128), F32),
            pltpu.VMEM((8 * SCH,), jnp.int32),
            pltpu.VMEM((1, SCH), jnp.int32),
            pltpu.VMEM((SCH, 128), F32),
            pltpu.VMEM((24, 128), F32),
            pltpu.SemaphoreType.DMA,
        ],
    )
    return f(dst1)


def _pool_body(x_hbm, bat_hbm, out_hbm, acc, batb, idxb, rb, zbuf,
               ssem):
    c = lax.axis_index("c")
    tid = lax.axis_index("s")

    @pl.when((c == 0) & (tid == 0))
    def _():
        _zfill(zbuf, 125)
        for z in range(8):
            pltpu.sync_copy(zbuf, acc.at[pl.ds(z * 125, 125)])

    plsc.subcore_barrier()

    @pl.when(c == 0)
    def _():
        def chunk(cid):
            pltpu.sync_copy(bat_hbm.at[pl.ds(cid * 200, 200)], batb)
            pltpu.sync_copy(x_hbm.at[pl.ds(cid * 200, 200)], rb)
            for i in range(2):
                for k in (0, 16, 32, 48, 64, 80, 84):
                    idxb[i, pl.ds(k, 16)] = batb[pl.ds(i * 100 + k, 16)]
            fires = []
            for i in range(2):
                fires.append(pltpu.async_copy(
                    rb.at[pl.ds(i * 100, 100)],
                    acc.at[idxb.at[i]], ssem, add=True))
            for fcp in fires:
                fcp.wait()

        def body(k, carry):
            chunk(k * 16 + tid)
            return carry

        lax.fori_loop(0, 15, body, 0)

        @pl.when(tid < 10)
        def _():
            chunk(240 + tid)

    plsc.subcore_barrier()

    @pl.when((c == 0) & (tid == 0))
    def _():
        pltpu.sync_copy(acc, out_hbm)


def _pool(x, bat1):
    f = pl.kernel(
        _pool_body,
        out_type=jax.ShapeDtypeStruct((NG, 128), F32),
        mesh=plsc.VectorSubcoreMesh(**_MESH),
        scratch_types=[
            pltpu.VMEM_SHARED((NG, 128), F32),
            pltpu.VMEM((200,), jnp.int32),
            pltpu.VMEM((2, 100), jnp.int32),
            pltpu.VMEM((200, 128), F32),
            pltpu.VMEM((125, 128), F32),
            pltpu.SemaphoreType.DMA,
        ],
    )
    return f(x, bat1)


def kernel(node_s, node_v, edge_s, edge_v, edge_index, batch, params):
    src = edge_index[0]
    dst = edge_index[1]

    h = _embed_nodes(node_s, node_v, params)
    eft = _embed_edges(edge_s, edge_v, params)
    cnt = _cnt(dst)
    for i in range(3):
        lp = params['layers'][i]
        gs, gd = _gather(h, src, dst)
        msgs = _messages(gs, gd, eft, lp['conv'])
        agg = _scatter(msgs, dst)
        h = _node_update(h, agg, cnt, lp)
    out50 = _readout(h, params)
    return _pool(out50, batch)
